# Initial kernel scaffold; baseline (speedup 1.0000x reference)
import jax
import jax.numpy as jnp
from jax.experimental import pallas as pl


def kernel(frac_coords, atom_types, lengths, angles, edge_index, to_jimages, num_atoms, num_bonds, params):
    raise NotImplementedError("write your pallas kernel here")


# speedup vs baseline: 2.2397x; 2.2397x over previous
import functools

import jax
import jax.numpy as jnp
from jax import lax
from jax.experimental import pallas as pl
from jax.experimental.pallas import tpu as pltpu
from jax.experimental.pallas import tpu_sc as plsc

N = 10000
E = 320000
B = 100
H = 128
T = 128
NRBF = 16
CUTOFF = 6.0
NTYPES = 100
NBLOCKS = 3

NP = 10240
NW = 32
EPW = E // NW
CH = 80
NCH = EPW // CH
RPS = NP // 16

ET = 512
NET = E // ET
F32 = jnp.float32


def _lat_from_params(lengths, angles):
    a, b, c = lengths[:, 0], lengths[:, 1], lengths[:, 2]
    ang = jnp.deg2rad(angles)
    cos_a, cos_b, cos_g = jnp.cos(ang[:, 0]), jnp.cos(ang[:, 1]), jnp.cos(ang[:, 2])
    sin_a, sin_b = jnp.sin(ang[:, 0]), jnp.sin(ang[:, 1])
    val = (cos_a * cos_b - cos_g) / jnp.clip(sin_a * sin_b, 1e-8, None)
    val = jnp.clip(val, -1.0, 1.0)
    gamma_star = jnp.arccos(val)
    zeros = jnp.zeros_like(a)
    va = jnp.stack([a * sin_b, zeros, a * cos_b], axis=-1)
    vb = jnp.stack([-b * sin_a * jnp.cos(gamma_star), b * sin_a * jnp.sin(gamma_star), b * cos_a], axis=-1)
    vc = jnp.stack([zeros, zeros, c], axis=-1)
    return jnp.stack([va, vb, vc], axis=1)


def _prep_body(at_ref, emb_ref, wej_ref, wei_ref, wmj_ref, wmi_ref,
               h0_ref, tej_ref, tei_ref, tmj_ref, tmi_ref):
    types = at_ref[0, 0, :]
    oh = jnp.equal(types[:, None],
                   lax.broadcasted_iota(jnp.int32, (H, H), 1)).astype(F32)
    h0 = jnp.dot(oh, emb_ref[...], preferred_element_type=F32)
    h0_ref[...] = h0
    tej_ref[...] = jnp.dot(h0, wej_ref[...], preferred_element_type=F32)
    tei_ref[...] = jnp.dot(h0, wei_ref[...], preferred_element_type=F32)
    tmj_ref[...] = jnp.dot(h0, wmj_ref[...], preferred_element_type=F32)
    tmi_ref[...] = jnp.dot(h0, wmi_ref[...], preferred_element_type=F32)


def _prep_call(at3, embP, wej, wei, wmj, wmi):
    nh = jax.ShapeDtypeStruct((NP, H), F32)
    full = pl.BlockSpec((H, H), lambda t: (0, 0))
    return pl.pallas_call(
        _prep_body,
        grid=(NP // H,),
        in_specs=[pl.BlockSpec((1, 1, H), lambda t: (t, 0, 0)),
                  full, full, full, full, full],
        out_specs=[pl.BlockSpec((H, H), lambda t: (t, 0))] * 5,
        out_shape=[nh] * 5,
    )(at3, embP, wej, wei, wmj, wmi)


def _einit_body(gj_ref, gi_ref, tji_ref, wer_ref, be_ref, e_ref):
    gj = gj_ref[...]
    gi = gi_ref[...]
    tji = tji_ref[...]
    d2 = None
    for k in range(3):
        off = (tji[:, 0:1] * gi[:, 131 + k:132 + k]
               + tji[:, 1:2] * gi[:, 134 + k:135 + k]
               + tji[:, 2:3] * gi[:, 137 + k:138 + k])
        dk = gj[:, 128 + k:129 + k] - gi[:, 128 + k:129 + k] + off
        d2 = dk * dk if d2 is None else d2 + dk * dk
    dist = jnp.sqrt(d2)
    step = CUTOFF / (NRBF - 1)
    centers = lax.broadcasted_iota(jnp.int32, (1, NRBF), 1).astype(F32) * step
    width = CUTOFF / NRBF
    rbf = jnp.exp(-((dist - centers) ** 2) * (1.0 / (2.0 * width * width)))
    pre = (gj[:, :H] + gi[:, :H]
           + jnp.dot(rbf, wer_ref[...], preferred_element_type=F32)
           + be_ref[...])
    e_ref[...] = jnp.tanh(pre)


def _einit_call(gj, gi, tji8, wer, be):
    return pl.pallas_call(
        _einit_body,
        grid=(NET,),
        in_specs=[pl.BlockSpec((ET, 144), lambda t: (t, 0)),
                  pl.BlockSpec((ET, 144), lambda t: (t, 0)),
                  pl.BlockSpec((ET, 8), lambda t: (t, 0)),
                  pl.BlockSpec((NRBF, H), lambda t: (0, 0)),
                  pl.BlockSpec((1, H), lambda t: (0, 0))],
        out_specs=pl.BlockSpec((ET, H), lambda t: (t, 0)),
        out_shape=jax.ShapeDtypeStruct((E, H), F32),
    )(gj, gi, tji8, wer, be)


def _blk_body(gj_ref, gi_ref, e_ref, wme_ref, bm_ref, we2_ref, be2_ref,
              m_ref, eo_ref):
    e = e_ref[...]
    m = jnp.tanh(gj_ref[...] + gi_ref[...]
                 + jnp.dot(e, wme_ref[...], preferred_element_type=F32)
                 + bm_ref[...])
    m_ref[...] = m
    eo_ref[...] = e + jnp.tanh(jnp.dot(m, we2_ref[...], preferred_element_type=F32)
                               + be2_ref[...])


def _blk_last_body(gj_ref, gi_ref, e_ref, wme_ref, bm_ref, m_ref):
    m = jnp.tanh(gj_ref[...] + gi_ref[...]
                 + jnp.dot(e_ref[...], wme_ref[...], preferred_element_type=F32)
                 + bm_ref[...])
    m_ref[...] = m


def _blk_call(gj, gi, e, wme, bm, we2, be2, last):
    eh = pl.BlockSpec((ET, H), lambda t: (t, 0))
    full = pl.BlockSpec((H, H), lambda t: (0, 0))
    bias = pl.BlockSpec((1, H), lambda t: (0, 0))
    esh = jax.ShapeDtypeStruct((E, H), F32)
    if last:
        return pl.pallas_call(
            _blk_last_body, grid=(NET,),
            in_specs=[eh, eh, eh, full, bias],
            out_specs=eh, out_shape=esh,
        )(gj, gi, e, wme, bm), None
    m, eo = pl.pallas_call(
        _blk_body, grid=(NET,),
        in_specs=[eh, eh, eh, full, bias, full, bias],
        out_specs=[eh, eh], out_shape=[esh, esh],
    )(gj, gi, e, wme, bm, we2, be2)
    return m, eo


def _node_body(agg_ref, h_ref, wh_ref, bh_ref, wnj_ref, wni_ref,
               h2_ref, tmj_ref, tmi_ref):
    agg = agg_ref[0] + agg_ref[1]
    h2 = h_ref[...] + jnp.tanh(jnp.dot(agg, wh_ref[...], preferred_element_type=F32)
                               + bh_ref[...])
    h2_ref[...] = h2
    tmj_ref[...] = jnp.dot(h2, wnj_ref[...], preferred_element_type=F32)
    tmi_ref[...] = jnp.dot(h2, wni_ref[...], preferred_element_type=F32)


def _node_last_body(agg_ref, h_ref, wh_ref, bh_ref, wout_ref, bout_ref, t_ref):
    agg = agg_ref[0] + agg_ref[1]
    h2 = h_ref[...] + jnp.tanh(jnp.dot(agg, wh_ref[...], preferred_element_type=F32)
                               + bh_ref[...])
    t_ref[...] = jnp.dot(h2, wout_ref[...], preferred_element_type=F32) + bout_ref[...]


def _node_call(agg2, h, wh, bh, wa, wb, last):
    nh = pl.BlockSpec((H, H), lambda t: (t, 0))
    full = pl.BlockSpec((H, H), lambda t: (0, 0))
    bias = pl.BlockSpec((1, H), lambda t: (0, 0))
    nsh = jax.ShapeDtypeStruct((NP, H), F32)
    aggspec = pl.BlockSpec((2, H, H), lambda t: (0, t, 0))
    if last:
        return pl.pallas_call(
            _node_last_body, grid=(NP // H,),
            in_specs=[aggspec, nh, full, bias, full, bias],
            out_specs=nh, out_shape=nsh,
        )(agg2, h, wh, bh, wa, wb)
    return pl.pallas_call(
        _node_body, grid=(NP // H,),
        in_specs=[aggspec, nh, full, bias, full, full],
        out_specs=[nh] * 3, out_shape=[nsh] * 3,
    )(agg2, h, wh, bh, wa, wb)


def _final_body(t_ref, s_ref, wmu_ref, bmu_ref, wvar_ref, bvar_ref,
                mu_ref, lv_ref, hid_ref):
    hid = jnp.dot(s_ref[...], t_ref[...], preferred_element_type=F32)
    hid_ref[...] = hid
    mu_ref[...] = jnp.dot(hid, wmu_ref[...], preferred_element_type=F32) + bmu_ref[...]
    x = jnp.dot(hid, wvar_ref[...], preferred_element_type=F32) + bvar_ref[...]
    lv_ref[...] = (jnp.maximum(x, 0.0) + jnp.log1p(jnp.exp(-jnp.abs(x))) + 1e-05)


def _final_call(t, sel, wmu, bmu, wvar, bvar):
    osh = jax.ShapeDtypeStruct((B, T), F32)
    return pl.pallas_call(
        _final_body,
        grid=(1,),
        in_specs=[pl.BlockSpec((NP, H), lambda i: (0, 0)),
                  pl.BlockSpec((B, NP), lambda i: (0, 0)),
                  pl.BlockSpec((T, T), lambda i: (0, 0)),
                  pl.BlockSpec((1, T), lambda i: (0, 0)),
                  pl.BlockSpec((T, T), lambda i: (0, 0)),
                  pl.BlockSpec((1, T), lambda i: (0, 0))],
        out_specs=[pl.BlockSpec((B, T), lambda i: (0, 0))] * 3,
        out_shape=[osh] * 3,
    )(t, sel, wmu, bmu, wvar, bvar)


def _sc_gather2(dj, di):
    mesh = plsc.VectorSubcoreMesh(core_axis_name="c", subcore_axis_name="s")

    @functools.partial(
        pl.kernel, mesh=mesh,
        compiler_params=pltpu.CompilerParams(use_tc_tiling_on_sc=False),
        out_type=[jax.ShapeDtypeStruct((E, dj), F32),
                  jax.ShapeDtypeStruct((E, di), F32)],
        scratch_types=[pltpu.VMEM((NCH, CH), jnp.int32),
                       pltpu.VMEM((NCH, CH), jnp.int32),
                       pltpu.VMEM((CH, dj), F32),
                       pltpu.VMEM((CH, di), F32),
                       pltpu.SemaphoreType.DMA,
                       pltpu.SemaphoreType.DMA],
    )
    def k(tabj, tabi, idxj, idxi, outj, outi, idxj_v, idxi_v, bufj, bufi,
          semj, semi):
        c = lax.axis_index("c")
        s = lax.axis_index("s")
        w = s * 2 + c
        pltpu.sync_copy(idxj.at[w], idxj_v)
        pltpu.sync_copy(idxi.at[w], idxi_v)

        def body(ci, _):
            cpj = pltpu.async_copy(tabj.at[idxj_v.at[ci]], bufj, semj)
            cpi = pltpu.async_copy(tabi.at[idxi_v.at[ci]], bufi, semi)
            cpj.wait()
            cpi.wait()
            base = w * EPW + ci * CH
            pltpu.sync_copy(bufj, outj.at[pl.ds(base, CH)])
            pltpu.sync_copy(bufi, outi.at[pl.ds(base, CH)])
            return 0

        lax.fori_loop(0, NCH, body, 0)

    return k


def _sc_scatter_add():
    mesh = plsc.VectorSubcoreMesh(core_axis_name="c", subcore_axis_name="s")

    @functools.partial(
        pl.kernel, mesh=mesh,
        compiler_params=pltpu.CompilerParams(use_tc_tiling_on_sc=False),
        out_type=jax.ShapeDtypeStruct((2, NP, H), F32),
        scratch_types=[pltpu.VMEM((NCH, CH), jnp.int32),
                       pltpu.VMEM((CH, H), F32),
                       pltpu.VMEM((CH, H), F32),
                       pltpu.VMEM_SHARED((NP, H), F32),
                       pltpu.SemaphoreType.DMA],
    )
    def k(m_hbm, idxi, out, idx_v, mbuf, zbuf, acc, sem):
        c = lax.axis_index("c")
        s = lax.axis_index("s")
        w = s * 2 + c

        def zrow(r, _):
            for q in range(H // 16):
                zbuf[r, pl.ds(q * 16, 16)] = jnp.zeros((16,), F32)
            return 0

        lax.fori_loop(0, CH, zrow, 0)

        def zblk(bi, _):
            pltpu.sync_copy(zbuf, acc.at[pl.ds(s * RPS + bi * CH, CH)])
            return 0

        lax.fori_loop(0, RPS // CH, zblk, 0)
        plsc.subcore_barrier()

        pltpu.sync_copy(idxi.at[w], idx_v)

        def body(ci, _):
            base = w * EPW + ci * CH
            pltpu.sync_copy(m_hbm.at[pl.ds(base, CH)], mbuf)
            pltpu.sync_copy(mbuf, acc.at[idx_v.at[ci]], add=True)
            return 0

        lax.fori_loop(0, NCH, body, 0)
        plsc.subcore_barrier()
        pltpu.sync_copy(acc.at[pl.ds(s * RPS, RPS)],
                        out.at[c, pl.ds(s * RPS, RPS)])

    return k


def kernel(frac_coords, atom_types, lengths, angles, edge_index, to_jimages,
           num_atoms, num_bonds, params):
    p = params
    lat = _lat_from_params(lengths, angles)
    batch_node = jnp.arange(N, dtype=jnp.int32) // (N // B)
    latn = lat.reshape(B, 9)[batch_node]
    pos = jnp.einsum('ni,nij->nj', frac_coords, lat[batch_node])

    at_p = jnp.zeros((NP,), jnp.int32).at[:N].set(atom_types.astype(jnp.int32))
    at3 = at_p.reshape(NP // H, 1, H)

    embP = jnp.zeros((H, H), F32).at[:NTYPES].set(p['emb'])
    wedge = p['W_edge']
    wej, wei, wer = wedge[:H], wedge[H:2 * H], wedge[2 * H:]
    bm = p['bm'].reshape(NBLOCKS, 1, H)
    be2 = p['be2'].reshape(NBLOCKS, 1, H)
    bh = p['bh'].reshape(NBLOCKS, 1, H)
    be = p['b_edge'].reshape(1, H)
    bout = p['b_out'].reshape(1, T)
    bmu = p['bmu'].reshape(1, T)
    bvar = p['bvar'].reshape(1, T)

    idxj3 = edge_index[0].astype(jnp.int32).reshape(NW, NCH, CH)
    idxi3 = edge_index[1].astype(jnp.int32).reshape(NW, NCH, CH)
    tji8 = jnp.concatenate([to_jimages.astype(F32), jnp.zeros((E, 5), F32)], axis=1)

    h0, tej, tei, tmj, tmi = _prep_call(
        at3, embP, wej, wei, p['Wm'][0][:H], p['Wm'][0][H:2 * H])

    geo_j = jnp.concatenate([pos, jnp.zeros((N, 13), F32)], axis=1)
    geo_i = jnp.concatenate([pos, latn, jnp.zeros((N, 4), F32)], axis=1)
    zpad = jnp.zeros((NP - N, 16), F32)
    tabj = jnp.concatenate([tej, jnp.concatenate([geo_j, zpad], 0)], axis=1)
    tabi = jnp.concatenate([tei, jnp.concatenate([geo_i, zpad], 0)], axis=1)

    gj, gi = _sc_gather2(144, 144)(tabj, tabi, idxj3, idxi3)

    e = _einit_call(gj, gi, tji8, wer, be)

    scat = _sc_scatter_add()
    h = h0
    for blk in range(NBLOCKS):
        gmj, gmi = _sc_gather2(H, H)(tmj, tmi, idxj3, idxi3)
        wm = p['Wm'][blk]
        m, e = _blk_call(gmj, gmi, e, wm[2 * H:], bm[blk],
                         p['We2'][blk], be2[blk], last=(blk == NBLOCKS - 1))
        agg2 = scat(m, idxi3)
        last = blk == NBLOCKS - 1
        if last:
            t = _node_call(agg2, h, p['Wh'][blk], bh[blk],
                           p['W_out'], bout, last=True)
        else:
            wmn = p['Wm'][blk + 1]
            h, tmj, tmi = _node_call(agg2, h, p['Wh'][blk], bh[blk],
                                     wmn[:H], wmn[H:2 * H], last=False)

    na = num_atoms.astype(F32)
    sel = (jnp.equal(batch_node[None, :], jnp.arange(B, dtype=jnp.int32)[:, None])
           .astype(F32) / na[:, None])
    sel = jnp.concatenate([sel, jnp.zeros((B, NP - N), F32)], axis=1)
    mu, log_var, hidden = _final_call(t, sel, p['Wmu'], bmu, p['Wvar'], bvar)
    return (mu, log_var, hidden)


# speedup vs baseline: 2.6204x; 1.1700x over previous
import functools

import jax
import jax.numpy as jnp
from jax import lax
from jax.experimental import pallas as pl
from jax.experimental.pallas import tpu as pltpu
from jax.experimental.pallas import tpu_sc as plsc

N = 10000
E = 320000
B = 100
H = 128
T = 128
NRBF = 16
CUTOFF = 6.0
NTYPES = 100
NBLOCKS = 3

NP = 10240
NW = 32
EPW = E // NW
CH = 100
NCH = EPW // CH
RPS = NP // 16
CHS = 50
NCHS = EPW // CHS
GW = 16

ET = 512
NET = E // ET
F32 = jnp.float32

_SC_PARAMS = pltpu.CompilerParams(use_tc_tiling_on_sc=False)


def _lat_from_params(lengths, angles):
    a, b, c = lengths[:, 0], lengths[:, 1], lengths[:, 2]
    ang = jnp.deg2rad(angles)
    cos_a, cos_b, cos_g = jnp.cos(ang[:, 0]), jnp.cos(ang[:, 1]), jnp.cos(ang[:, 2])
    sin_a, sin_b = jnp.sin(ang[:, 0]), jnp.sin(ang[:, 1])
    val = (cos_a * cos_b - cos_g) / jnp.clip(sin_a * sin_b, 1e-8, None)
    val = jnp.clip(val, -1.0, 1.0)
    gamma_star = jnp.arccos(val)
    zeros = jnp.zeros_like(a)
    va = jnp.stack([a * sin_b, zeros, a * cos_b], axis=-1)
    vb = jnp.stack([-b * sin_a * jnp.cos(gamma_star), b * sin_a * jnp.sin(gamma_star), b * cos_a], axis=-1)
    vc = jnp.stack([zeros, zeros, c], axis=-1)
    return jnp.stack([va, vb, vc], axis=1)


def _prep_body(at_ref, emb_ref, wej_ref, wei_ref, wmj_ref, wmi_ref,
               h0_ref, tej_ref, tei_ref, tmj_ref, tmi_ref):
    types = at_ref[0, 0, :]
    oh = jnp.equal(types[:, None],
                   lax.broadcasted_iota(jnp.int32, (H, H), 1)).astype(F32)
    h0 = jnp.dot(oh, emb_ref[...], preferred_element_type=F32)
    h0_ref[...] = h0
    tej_ref[...] = jnp.dot(h0, wej_ref[...], preferred_element_type=F32)
    tei_ref[...] = jnp.dot(h0, wei_ref[...], preferred_element_type=F32)
    tmj_ref[...] = jnp.dot(h0, wmj_ref[...], preferred_element_type=F32)
    tmi_ref[...] = jnp.dot(h0, wmi_ref[...], preferred_element_type=F32)


def _prep_call(at3, embP, wej, wei, wmj, wmi):
    nh = jax.ShapeDtypeStruct((NP, H), F32)
    full = pl.BlockSpec((H, H), lambda t: (0, 0))
    return pl.pallas_call(
        _prep_body,
        grid=(NP // H,),
        in_specs=[pl.BlockSpec((1, 1, H), lambda t: (t, 0, 0)),
                  full, full, full, full, full],
        out_specs=[pl.BlockSpec((H, H), lambda t: (t, 0))] * 5,
        out_shape=[nh] * 5,
    )(at3, embP, wej, wei, wmj, wmi)


def _einit_body(gj_ref, gi_ref, pj_ref, pi_ref, tji_ref, wer_ref, be_ref, e_ref):
    pj = pj_ref[...]
    pi = pi_ref[...]
    tji = tji_ref[...]
    d2 = None
    for k in range(3):
        off = (tji[:, 0:1] * pi[:, 3 + k:4 + k]
               + tji[:, 1:2] * pi[:, 6 + k:7 + k]
               + tji[:, 2:3] * pi[:, 9 + k:10 + k])
        dk = pj[:, k:k + 1] - pi[:, k:k + 1] + off
        d2 = dk * dk if d2 is None else d2 + dk * dk
    dist = jnp.sqrt(d2)
    step = CUTOFF / (NRBF - 1)
    centers = lax.broadcasted_iota(jnp.int32, (1, NRBF), 1).astype(F32) * step
    width = CUTOFF / NRBF
    rbf = jnp.exp(-((dist - centers) ** 2) * (1.0 / (2.0 * width * width)))
    pre = (gj_ref[...] + gi_ref[...]
           + jnp.dot(rbf, wer_ref[...], preferred_element_type=F32)
           + be_ref[...])
    e_ref[...] = jnp.tanh(pre)


def _einit_call(gj, gi, pj, pi, tji8, wer, be):
    eh = pl.BlockSpec((ET, H), lambda t: (t, 0))
    return pl.pallas_call(
        _einit_body,
        grid=(NET,),
        in_specs=[eh, eh,
                  pl.BlockSpec((ET, GW), lambda t: (t, 0)),
                  pl.BlockSpec((ET, GW), lambda t: (t, 0)),
                  pl.BlockSpec((ET, 8), lambda t: (t, 0)),
                  pl.BlockSpec((NRBF, H), lambda t: (0, 0)),
                  pl.BlockSpec((1, H), lambda t: (0, 0))],
        out_specs=eh,
        out_shape=jax.ShapeDtypeStruct((E, H), F32),
    )(gj, gi, pj, pi, tji8, wer, be)


def _blk_body(gj_ref, gi_ref, e_ref, wme_ref, bm_ref, we2_ref, be2_ref,
              m_ref, eo_ref):
    e = e_ref[...]
    m = jnp.tanh(gj_ref[...] + gi_ref[...]
                 + jnp.dot(e, wme_ref[...], preferred_element_type=F32)
                 + bm_ref[...])
    m_ref[...] = m
    eo_ref[...] = e + jnp.tanh(jnp.dot(m, we2_ref[...], preferred_element_type=F32)
                               + be2_ref[...])


def _blk_last_body(gj_ref, gi_ref, e_ref, wme_ref, bm_ref, m_ref):
    m = jnp.tanh(gj_ref[...] + gi_ref[...]
                 + jnp.dot(e_ref[...], wme_ref[...], preferred_element_type=F32)
                 + bm_ref[...])
    m_ref[...] = m


def _blk_call(gj, gi, e, wme, bm, we2, be2, last):
    eh = pl.BlockSpec((ET, H), lambda t: (t, 0))
    full = pl.BlockSpec((H, H), lambda t: (0, 0))
    bias = pl.BlockSpec((1, H), lambda t: (0, 0))
    esh = jax.ShapeDtypeStruct((E, H), F32)
    if last:
        return pl.pallas_call(
            _blk_last_body, grid=(NET,),
            in_specs=[eh, eh, eh, full, bias],
            out_specs=eh, out_shape=esh,
        )(gj, gi, e, wme, bm), None
    m, eo = pl.pallas_call(
        _blk_body, grid=(NET,),
        in_specs=[eh, eh, eh, full, bias, full, bias],
        out_specs=[eh, eh], out_shape=[esh, esh],
    )(gj, gi, e, wme, bm, we2, be2)
    return m, eo


def _node_body(agg_ref, h_ref, wh_ref, bh_ref, wnj_ref, wni_ref,
               h2_ref, tmj_ref, tmi_ref):
    agg = agg_ref[0] + agg_ref[1]
    h2 = h_ref[...] + jnp.tanh(jnp.dot(agg, wh_ref[...], preferred_element_type=F32)
                               + bh_ref[...])
    h2_ref[...] = h2
    tmj_ref[...] = jnp.dot(h2, wnj_ref[...], preferred_element_type=F32)
    tmi_ref[...] = jnp.dot(h2, wni_ref[...], preferred_element_type=F32)


def _node_last_body(agg_ref, h_ref, wh_ref, bh_ref, wout_ref, bout_ref, t_ref):
    agg = agg_ref[0] + agg_ref[1]
    h2 = h_ref[...] + jnp.tanh(jnp.dot(agg, wh_ref[...], preferred_element_type=F32)
                               + bh_ref[...])
    t_ref[...] = jnp.dot(h2, wout_ref[...], preferred_element_type=F32) + bout_ref[...]


def _node_call(agg2, h, wh, bh, wa, wb, last):
    nh = pl.BlockSpec((H, H), lambda t: (t, 0))
    full = pl.BlockSpec((H, H), lambda t: (0, 0))
    bias = pl.BlockSpec((1, H), lambda t: (0, 0))
    nsh = jax.ShapeDtypeStruct((NP, H), F32)
    aggspec = pl.BlockSpec((2, H, H), lambda t: (0, t, 0))
    if last:
        return pl.pallas_call(
            _node_last_body, grid=(NP // H,),
            in_specs=[aggspec, nh, full, bias, full, bias],
            out_specs=nh, out_shape=nsh,
        )(agg2, h, wh, bh, wa, wb)
    return pl.pallas_call(
        _node_body, grid=(NP // H,),
        in_specs=[aggspec, nh, full, bias, full, full],
        out_specs=[nh] * 3, out_shape=[nsh] * 3,
    )(agg2, h, wh, bh, wa, wb)


def _final_body(t_ref, s_ref, wmu_ref, bmu_ref, wvar_ref, bvar_ref,
                mu_ref, lv_ref, hid_ref):
    hid = jnp.dot(s_ref[...], t_ref[...], preferred_element_type=F32)
    hid_ref[...] = hid
    mu_ref[...] = jnp.dot(hid, wmu_ref[...], preferred_element_type=F32) + bmu_ref[...]
    x = jnp.dot(hid, wvar_ref[...], preferred_element_type=F32) + bvar_ref[...]
    lv_ref[...] = (jnp.maximum(x, 0.0) + jnp.log1p(jnp.exp(-jnp.abs(x))) + 1e-05)


def _final_call(t, sel, wmu, bmu, wvar, bvar):
    osh = jax.ShapeDtypeStruct((B, T), F32)
    return pl.pallas_call(
        _final_body,
        grid=(1,),
        in_specs=[pl.BlockSpec((NP, H), lambda i: (0, 0)),
                  pl.BlockSpec((B, NP), lambda i: (0, 0)),
                  pl.BlockSpec((T, T), lambda i: (0, 0)),
                  pl.BlockSpec((1, T), lambda i: (0, 0)),
                  pl.BlockSpec((T, T), lambda i: (0, 0)),
                  pl.BlockSpec((1, T), lambda i: (0, 0))],
        out_specs=[pl.BlockSpec((B, T), lambda i: (0, 0))] * 3,
        out_shape=[osh] * 3,
    )(t, sel, wmu, bmu, wvar, bvar)


def _sc_gather_multi(widths, sels, M, D):
    n = len(widths)
    mesh = plsc.VectorSubcoreMesh(core_axis_name="c", subcore_axis_name="s")
    scratch = [pltpu.VMEM((NCH, CH), jnp.int32),
               pltpu.VMEM((NCH, CH), jnp.int32)]
    for _ in range(M):
        for t in range(n):
            scratch.append(pltpu.VMEM((CH, widths[t]), F32))
    scratch += [pltpu.SemaphoreType.DMA] * (2 * M)

    @functools.partial(
        pl.kernel, mesh=mesh, compiler_params=_SC_PARAMS,
        out_type=[jax.ShapeDtypeStruct((E, wd), F32) for wd in widths],
        scratch_types=scratch,
    )
    def k(*refs):
        tabs = refs[:n]
        idxj, idxi = refs[n], refs[n + 1]
        outs = refs[n + 2:2 * n + 2]
        sc = refs[2 * n + 2:]
        idxjv, idxiv = sc[0], sc[1]
        bufs = [[sc[2 + b * n + t] for t in range(n)] for b in range(M)]
        gsem = sc[2 + M * n:2 + M * n + M]
        wsem = sc[2 + M * n + M:]
        cx = lax.axis_index("c")
        sx = lax.axis_index("s")
        w = sx * 2 + cx
        pltpu.sync_copy(idxj.at[w], idxjv)
        pltpu.sync_copy(idxi.at[w], idxiv)
        idxsel = [idxjv if s_ == 'j' else idxiv for s_ in sels]

        def issue_p1(c, b):
            for t in range(n):
                pltpu.async_copy(tabs[t].at[idxsel[t].at[c]], bufs[b][t], gsem[b])

        def wait_p1(b):
            for t in range(n):
                pltpu.make_async_copy(tabs[t].at[idxsel[t].at[0]],
                                      bufs[b][t], gsem[b]).wait()

        def issue_p2(c, b):
            base = w * EPW + c * CH
            for t in range(n):
                pltpu.async_copy(bufs[b][t], outs[t].at[pl.ds(base, CH)], wsem[b])

        def wait_p2(b):
            for t in range(n):
                pltpu.make_async_copy(outs[t].at[pl.ds(0, CH)],
                                      bufs[b][t], wsem[b]).wait()

        for b in range(M):
            issue_p1(b, b)

        def group(g, _):
            for bb in range(M):
                c = g * M + bb
                wait_p1(bb)
                issue_p2(c, bb)
                cr = c - D
                crb = (bb - D) % M

                @pl.when(jnp.logical_and(cr >= 0, cr + M < NCH))
                def _():
                    wait_p2(crb)
                    issue_p1(cr + M, crb)
            return 0

        lax.fori_loop(0, NCH // M, group, 0)
        for b in range(M):
            wait_p2(b)

    return k


def _sc_scatter_add(M, D):
    mesh = plsc.VectorSubcoreMesh(core_axis_name="c", subcore_axis_name="s")
    scratch = [pltpu.VMEM((NCHS, CHS), jnp.int32)]
    scratch += [pltpu.VMEM((CHS, H), F32) for _ in range(M)]
    scratch += [pltpu.SemaphoreType.DMA] * (2 * M)

    @functools.partial(
        pl.kernel, mesh=mesh, compiler_params=_SC_PARAMS,
        out_type=jax.ShapeDtypeStruct((2, NP, H), F32),
        scratch_types=scratch + [pltpu.VMEM_SHARED((NP, H), F32)],
    )
    def k(m_hbm, idxi, out, *sc):
        idxv = sc[0]
        bufs = sc[1:1 + M]
        lsem = sc[1 + M:1 + 2 * M]
        ssem = sc[1 + 2 * M:1 + 3 * M]
        acc = sc[1 + 3 * M]
        cx = lax.axis_index("c")
        sx = lax.axis_index("s")
        w = sx * 2 + cx

        def zrow(r, _):
            for q in range(H // 16):
                bufs[0][r, pl.ds(q * 16, 16)] = jnp.zeros((16,), F32)
            return 0

        lax.fori_loop(0, CHS, zrow, 0)

        def zblk(bi, _):
            pltpu.sync_copy(bufs[0], acc.at[pl.ds(sx * RPS + bi * CHS, CHS)])
            return 0

        lax.fori_loop(0, RPS // CHS, zblk, 0)
        rem = RPS - (RPS // CHS) * CHS
        if rem:
            pltpu.sync_copy(bufs[0].at[pl.ds(0, rem)],
                            acc.at[pl.ds(sx * RPS + (RPS // CHS) * CHS, rem)])
        plsc.subcore_barrier()

        pltpu.sync_copy(idxi.at[w], idxv)

        def issue_p1(c, b):
            base = w * EPW + c * CHS
            pltpu.async_copy(m_hbm.at[pl.ds(base, CHS)], bufs[b], lsem[b])

        def wait_p1(b):
            pltpu.make_async_copy(m_hbm.at[pl.ds(0, CHS)], bufs[b], lsem[b]).wait()

        def issue_p2(c, b):
            pltpu.async_copy(bufs[b], acc.at[idxv.at[c]], ssem[b], add=True)

        def wait_p2(b):
            pltpu.make_async_copy(m_hbm.at[pl.ds(0, CHS)], bufs[b], ssem[b]).wait()

        for b in range(M):
            issue_p1(b, b)

        def group(g, _):
            for bb in range(M):
                c = g * M + bb
                wait_p1(bb)
                issue_p2(c, bb)
                cr = c - D
                crb = (bb - D) % M

                @pl.when(jnp.logical_and(cr >= 0, cr + M < NCHS))
                def _():
                    wait_p2(crb)
                    issue_p1(cr + M, crb)
            return 0

        lax.fori_loop(0, NCHS // M, group, 0)
        for b in range(M):
            wait_p2(b)

        plsc.subcore_barrier()
        pltpu.sync_copy(acc.at[pl.ds(sx * RPS, RPS)],
                        out.at[cx, pl.ds(sx * RPS, RPS)])

    return k


def kernel(frac_coords, atom_types, lengths, angles, edge_index, to_jimages,
           num_atoms, num_bonds, params):
    p = params
    lat = _lat_from_params(lengths, angles)
    batch_node = jnp.arange(N, dtype=jnp.int32) // (N // B)
    latn = lat.reshape(B, 9)[batch_node]
    pos = jnp.einsum('ni,nij->nj', frac_coords, lat[batch_node])

    at_p = jnp.zeros((NP,), jnp.int32).at[:N].set(atom_types.astype(jnp.int32))
    at3 = at_p.reshape(NP // H, 1, H)

    embP = jnp.zeros((H, H), F32).at[:NTYPES].set(p['emb'])
    wedge = p['W_edge']
    wej, wei, wer = wedge[:H], wedge[H:2 * H], wedge[2 * H:]
    bm = p['bm'].reshape(NBLOCKS, 1, H)
    be2 = p['be2'].reshape(NBLOCKS, 1, H)
    bh = p['bh'].reshape(NBLOCKS, 1, H)
    be = p['b_edge'].reshape(1, H)
    bout = p['b_out'].reshape(1, T)
    bmu = p['bmu'].reshape(1, T)
    bvar = p['bvar'].reshape(1, T)

    idxj3 = edge_index[0].astype(jnp.int32).reshape(NW, NCH, CH)
    idxi3 = edge_index[1].astype(jnp.int32).reshape(NW, NCH, CH)
    idxi3s = edge_index[1].astype(jnp.int32).reshape(NW, NCHS, CHS)
    tji8 = jnp.concatenate([to_jimages.astype(F32), jnp.zeros((E, 5), F32)], axis=1)

    zg = jnp.zeros((NP - N, GW), F32)
    geoj = jnp.concatenate(
        [jnp.concatenate([pos, jnp.zeros((N, GW - 3), F32)], axis=1), zg], axis=0)
    geoi = jnp.concatenate(
        [jnp.concatenate([pos, latn, jnp.zeros((N, GW - 12), F32)], axis=1), zg],
        axis=0)

    h0, tej, tei, tmj, tmi = _prep_call(
        at3, embP, wej, wei, p['Wm'][0][:H], p['Wm'][0][H:2 * H])

    gj, gi, pgj, pgi = _sc_gather_multi((H, H, GW, GW), 'jiji', 2, 1)(
        tej, tei, geoj, geoi, idxj3, idxi3)

    e = _einit_call(gj, gi, pgj, pgi, tji8, wer, be)

    gat2 = _sc_gather_multi((H, H), 'ji', 4, 2)
    scat = _sc_scatter_add(4, 2)
    h = h0
    for blk in range(NBLOCKS):
        gmj, gmi = gat2(tmj, tmi, idxj3, idxi3)
        wm = p['Wm'][blk]
        m, e = _blk_call(gmj, gmi, e, wm[2 * H:], bm[blk],
                         p['We2'][blk], be2[blk], last=(blk == NBLOCKS - 1))
        agg2 = scat(m, idxi3s)
        last = blk == NBLOCKS - 1
        if last:
            t = _node_call(agg2, h, p['Wh'][blk], bh[blk],
                           p['W_out'], bout, last=True)
        else:
            wmn = p['Wm'][blk + 1]
            h, tmj, tmi = _node_call(agg2, h, p['Wh'][blk], bh[blk],
                                     wmn[:H], wmn[H:2 * H], last=False)

    na = num_atoms.astype(F32)
    sel = (jnp.equal(batch_node[None, :], jnp.arange(B, dtype=jnp.int32)[:, None])
           .astype(F32) / na[:, None])
    sel = jnp.concatenate([sel, jnp.zeros((B, NP - N), F32)], axis=1)
    mu, log_var, hidden = _final_call(t, sel, p['Wmu'], bmu, p['Wvar'], bvar)
    return (mu, log_var, hidden)
